# Initial kernel scaffold; baseline (speedup 1.0000x reference)
import functools

import jax
import jax.numpy as jnp
from jax import lax
from jax.experimental import pallas as pl
from jax.experimental.pallas import tpu as pltpu
from jax.experimental.pallas import tpu_sc as plsc

B, T, F = 4096, 200, 64
NC, NS, L = 2, 16, 16
NW = NC * NS
RPW = B // NW


def _sc_body(flat_hbm, out_hbm, idx_v, rows_v, tbl_v, sem):
    wid = lax.axis_index("s") * NC + lax.axis_index("c")
    base = wid * RPW

    tbl_v[...] = lax.iota(jnp.int32, (L,))

    def fill(j, carry):
        rows = lax.iota(jnp.int32, (L,)) + (base + j * L)
        idx_v[pl.ds(j * L, L)] = rows * T + (T - 1)
        return carry

    lax.fori_loop(0, RPW // L, fill, 0)

    pltpu.async_copy(flat_hbm.at[idx_v], rows_v, sem).wait()

    def body(i, carry):
        r = i // (F // L)
        c = lax.rem(i, F // L) * L
        vals = rows_v[r, pl.ds(c, L)]
        rows_v[r, pl.ds(c, L)] = plsc.load_gather(tbl_v, [vals])
        return carry

    lax.fori_loop(0, RPW * (F // L), body, 0)

    pltpu.sync_copy(rows_v, out_hbm.at[pl.ds(base, RPW)])


@jax.jit
def kernel(indices):
    flat = indices.reshape(B * T, F)
    run = pl.kernel(
        _sc_body,
        out_type=jax.ShapeDtypeStruct((B, F), jnp.int32),
        mesh=plsc.VectorSubcoreMesh(core_axis_name="c", subcore_axis_name="s"),
        scratch_types=[
            pltpu.VMEM((RPW,), jnp.int32),
            pltpu.VMEM((RPW, F), jnp.int32),
            pltpu.VMEM((L,), jnp.int32),
            pltpu.SemaphoreType.DMA,
        ],
    )
    return run(flat)


# speedup vs baseline: 3.5738x; 3.5738x over previous
import functools

import jax
import jax.numpy as jnp
from jax import lax
from jax.experimental import pallas as pl
from jax.experimental.pallas import tpu as pltpu
from jax.experimental.pallas import tpu_sc as plsc

B, T, F = 4096, 200, 64
NC, NS, L = 2, 16, 16
NW = NC * NS
RPW = B // NW


def _sc_body(flat_hbm, out_hbm, idx_v, rows_v, sem):
    wid = lax.axis_index("s") * NC + lax.axis_index("c")
    base = wid * RPW

    def fill(j, carry):
        rows = lax.iota(jnp.int32, L) + (base + j * L)
        idx_v[pl.ds(j * L, L)] = rows * T + (T - 1)
        return carry

    lax.fori_loop(0, RPW // L, fill, 0)

    pltpu.async_copy(flat_hbm.at[idx_v], rows_v, sem).wait()

    table_vals = tuple(range(5))

    def body(i, carry):
        r = i // (F // L)
        c = lax.rem(i, F // L) * L
        idx = rows_v[r, pl.ds(c, L)]
        out = jnp.full((L,), table_vals[0], dtype=jnp.int32)
        for k in range(1, len(table_vals)):
            out = jnp.where(idx == k, jnp.int32(table_vals[k]), out)
        rows_v[r, pl.ds(c, L)] = out
        return carry

    lax.fori_loop(0, RPW * (F // L), body, 0)

    pltpu.sync_copy(rows_v, out_hbm.at[pl.ds(base, RPW)])


@jax.jit
def kernel(indices):
    flat = indices.reshape(B * T, F)
    run = pl.kernel(
        _sc_body,
        out_type=jax.ShapeDtypeStruct((B, F), jnp.int32),
        mesh=plsc.VectorSubcoreMesh(core_axis_name="c", subcore_axis_name="s"),
        compiler_params=pltpu.CompilerParams(use_tc_tiling_on_sc=False),
        scratch_types=[
            pltpu.VMEM((RPW,), jnp.int32),
            pltpu.VMEM((RPW, F), jnp.int32),
            pltpu.SemaphoreType.DMA,
        ],
    )
    return run(flat)


# speedup vs baseline: 5.6971x; 1.5941x over previous
import functools

import jax
import jax.numpy as jnp
from jax import lax
from jax.experimental import pallas as pl
from jax.experimental.pallas import tpu as pltpu
from jax.experimental.pallas import tpu_sc as plsc

B, T, F = 4096, 200, 64
NC, NS, L = 2, 16, 16
NW = NC * NS
RPW = B // NW


def _sc_body(in_hbm, out_hbm, rows_v, sem):
    wid = lax.axis_index("s") * NC + lax.axis_index("c")
    base = wid * RPW

    pltpu.async_copy(in_hbm.at[pl.ds(base, RPW), T - 1, :], rows_v, sem).wait()

    table_vals = tuple(range(5))

    def body(i, carry):
        r = i // (F // L)
        c = lax.rem(i, F // L) * L
        idx = rows_v[r, pl.ds(c, L)]
        out = jnp.full((L,), table_vals[0], dtype=jnp.int32)
        for k in range(1, len(table_vals)):
            out = jnp.where(idx == k, jnp.int32(table_vals[k]), out)
        rows_v[r, pl.ds(c, L)] = out
        return carry

    lax.fori_loop(0, RPW * (F // L), body, 0)

    pltpu.sync_copy(rows_v, out_hbm.at[pl.ds(base, RPW)])


@jax.jit
def kernel(indices):
    run = pl.kernel(
        _sc_body,
        out_type=jax.ShapeDtypeStruct((B, F), jnp.int32),
        mesh=plsc.VectorSubcoreMesh(core_axis_name="c", subcore_axis_name="s"),
        scratch_types=[
            pltpu.VMEM((RPW, F), jnp.int32),
            pltpu.SemaphoreType.DMA,
        ],
    )
    return run(indices)


# speedup vs baseline: 74.6533x; 13.1038x over previous
import functools

import jax
import jax.numpy as jnp
from jax import lax
from jax.experimental import pallas as pl
from jax.experimental.pallas import tpu as pltpu
from jax.experimental.pallas import tpu_sc as plsc

B, T, F = 4096, 200, 64
NC, NS, L = 2, 16, 16
NW = NC * NS
CPW = B // NW


def _sc_body(in_hbm, out_hbm, slab_v, sem):
    wid = lax.axis_index("s") * NC + lax.axis_index("c")
    base = wid * CPW

    pltpu.async_copy(in_hbm.at[T - 1, :, pl.ds(base, CPW)], slab_v, sem).wait()

    table_vals = tuple(range(5))

    def body(i, carry):
        r = i // (CPW // L)
        c = lax.rem(i, CPW // L) * L
        idx = slab_v[r, pl.ds(c, L)]
        out = jnp.full((L,), table_vals[0], dtype=jnp.int32)
        for k in range(1, len(table_vals)):
            out = jnp.where(idx == k, jnp.int32(table_vals[k]), out)
        slab_v[r, pl.ds(c, L)] = out
        return carry

    lax.fori_loop(0, F * (CPW // L), body, 0)

    pltpu.sync_copy(slab_v, out_hbm.at[:, pl.ds(base, CPW)])


@jax.jit
def kernel(indices):
    tview = jnp.transpose(indices, (1, 2, 0))
    run = pl.kernel(
        _sc_body,
        out_type=jax.ShapeDtypeStruct((F, B), jnp.int32),
        mesh=plsc.VectorSubcoreMesh(core_axis_name="c", subcore_axis_name="s"),
        scratch_types=[
            pltpu.VMEM((F, CPW), jnp.int32),
            pltpu.SemaphoreType.DMA,
        ],
    )
    return jnp.transpose(run(tview))


# speedup vs baseline: 75.5963x; 1.0126x over previous
import functools

import jax
import jax.numpy as jnp
from jax import lax
from jax.experimental import pallas as pl
from jax.experimental.pallas import tpu as pltpu
from jax.experimental.pallas import tpu_sc as plsc

B, T, F = 4096, 200, 64
NC, NS, L = 2, 16, 16
NW = NC * NS
CPW = B // NW


def _sc_body(in_hbm, out_hbm, slab_v, sem):
    wid = lax.axis_index("s") * NC + lax.axis_index("c")
    base = wid * CPW

    pltpu.async_copy(in_hbm.at[T - 1, :, pl.ds(base, CPW)], slab_v, sem).wait()

    table = lax.iota(jnp.int32, L)

    def body(i, carry):
        r = i // (CPW // L)
        c = lax.rem(i, CPW // L) * L
        idx = slab_v[r, pl.ds(c, L)]
        slab_v[r, pl.ds(c, L)] = lax.gather(
            table,
            idx[:, None],
            lax.GatherDimensionNumbers(
                offset_dims=(),
                collapsed_slice_dims=(0,),
                start_index_map=(0,),
            ),
            slice_sizes=(1,),
            mode=lax.GatherScatterMode.PROMISE_IN_BOUNDS,
        )
        return carry

    lax.fori_loop(0, F * (CPW // L), body, 0)

    pltpu.sync_copy(slab_v, out_hbm.at[:, pl.ds(base, CPW)])


@jax.jit
def kernel(indices):
    tview = jnp.transpose(indices, (1, 2, 0))
    run = pl.kernel(
        _sc_body,
        out_type=jax.ShapeDtypeStruct((F, B), jnp.int32),
        mesh=plsc.VectorSubcoreMesh(core_axis_name="c", subcore_axis_name="s"),
        scratch_types=[
            pltpu.VMEM((F, CPW), jnp.int32),
            pltpu.SemaphoreType.DMA,
        ],
    )
    return jnp.transpose(run(tview))


# speedup vs baseline: 84.7444x; 1.1210x over previous
import functools

import jax
import jax.numpy as jnp
from jax import lax
from jax.experimental import pallas as pl
from jax.experimental.pallas import tpu as pltpu
from jax.experimental.pallas import tpu_sc as plsc

B, T, F = 4096, 200, 64
NC, NS, L = 2, 16, 16
NW = NC * NS
CPW = B // NW


def _sc_body(in_hbm, out_hbm, slab_v, sem):
    wid = lax.axis_index("s") * NC + lax.axis_index("c")
    base = wid * CPW

    pltpu.async_copy(in_hbm.at[T - 1, :, pl.ds(base, CPW)], slab_v, sem).wait()

    table = lax.iota(jnp.int32, L)

    dnums = lax.GatherDimensionNumbers(
        offset_dims=(), collapsed_slice_dims=(0,), start_index_map=(0,)
    )

    def body(r, carry):
        for j in range(CPW // L):
            idx = slab_v[r, pl.ds(j * L, L)]
            slab_v[r, pl.ds(j * L, L)] = lax.gather(
                table, idx[:, None], dnums, slice_sizes=(1,),
                mode=lax.GatherScatterMode.PROMISE_IN_BOUNDS,
            )
        return carry

    lax.fori_loop(0, F, body, 0)

    pltpu.sync_copy(slab_v, out_hbm.at[:, pl.ds(base, CPW)])


@jax.jit
def kernel(indices):
    tview = jnp.transpose(indices, (1, 2, 0))
    run = pl.kernel(
        _sc_body,
        out_type=jax.ShapeDtypeStruct((F, B), jnp.int32),
        mesh=plsc.VectorSubcoreMesh(core_axis_name="c", subcore_axis_name="s"),
        scratch_types=[
            pltpu.VMEM((F, CPW), jnp.int32),
            pltpu.SemaphoreType.DMA,
        ],
    )
    return jnp.transpose(run(tview))


# speedup vs baseline: 85.0653x; 1.0038x over previous
import functools

import jax
import jax.numpy as jnp
from jax import lax
from jax.experimental import pallas as pl
from jax.experimental.pallas import tpu as pltpu
from jax.experimental.pallas import tpu_sc as plsc

B, T, F = 4096, 200, 64
NC, NS, L = 2, 16, 16
NW = NC * NS
CPW = B // NW


def _sc_body(in_hbm, out_hbm, slab_v, sem0, sem1, osem):
    wid = lax.axis_index("s") * NC + lax.axis_index("c")
    base = wid * CPW
    H = F // 2

    cp0 = pltpu.async_copy(
        in_hbm.at[T - 1, pl.ds(0, H), pl.ds(base, CPW)], slab_v.at[pl.ds(0, H)], sem0
    )
    cp1 = pltpu.async_copy(
        in_hbm.at[T - 1, pl.ds(H, H), pl.ds(base, CPW)], slab_v.at[pl.ds(H, H)], sem1
    )

    table = lax.iota(jnp.int32, L)
    dnums = lax.GatherDimensionNumbers(
        offset_dims=(), collapsed_slice_dims=(0,), start_index_map=(0,)
    )

    def body(r, carry):
        for j in range(CPW // L):
            idx = slab_v[r, pl.ds(j * L, L)]
            slab_v[r, pl.ds(j * L, L)] = lax.gather(
                table, idx[:, None], dnums, slice_sizes=(1,),
                mode=lax.GatherScatterMode.PROMISE_IN_BOUNDS,
            )
        return carry

    cp0.wait()
    lax.fori_loop(0, H, body, 0)
    ocp0 = pltpu.async_copy(
        slab_v.at[pl.ds(0, H)], out_hbm.at[pl.ds(0, H), pl.ds(base, CPW)], osem
    )
    cp1.wait()
    lax.fori_loop(H, F, body, 0)
    ocp1 = pltpu.async_copy(
        slab_v.at[pl.ds(H, H)], out_hbm.at[pl.ds(H, H), pl.ds(base, CPW)], osem
    )
    ocp0.wait()
    ocp1.wait()


@jax.jit
def kernel(indices):
    tview = jnp.transpose(indices, (1, 2, 0))
    run = pl.kernel(
        _sc_body,
        out_type=jax.ShapeDtypeStruct((F, B), jnp.int32),
        mesh=plsc.VectorSubcoreMesh(core_axis_name="c", subcore_axis_name="s"),
        scratch_types=[
            pltpu.VMEM((F, CPW), jnp.int32),
            pltpu.SemaphoreType.DMA,
            pltpu.SemaphoreType.DMA,
            pltpu.SemaphoreType.DMA,
        ],
    )
    return jnp.transpose(run(tview))
